# baseline (speedup 1.0000x reference)
import functools

import jax
import jax.numpy as jnp
from jax import lax
from jax.experimental import pallas as pl
from jax.experimental.pallas import tpu as pltpu
from jax.experimental.pallas import tpu_sc as plsc

_NW = 32
_LANES = 16


def _wid():
    return lax.axis_index("s") * 2 + lax.axis_index("c")


def _splat(value):
    return jnp.full((_LANES,), value, dtype=jnp.int32)


@functools.lru_cache(maxsize=None)
def _make_lookup(B, S, D, V, BC):
    assert BC == 128 and D == 64 and S % 8 == 0 and B % BC == 0 and V % 2 == 0
    chunks = B // BC
    n_tasks = (S // 8) * chunks
    assert n_tasks % _NW == 0
    n_g = BC // _LANES

    mesh = plsc.VectorSubcoreMesh(core_axis_name="c", subcore_axis_name="s")

    @functools.partial(
        pl.kernel,
        mesh=mesh,
        out_type=jax.ShapeDtypeStruct((S, D, B), jnp.float32),
        scratch_types=[
            pltpu.VMEM((8, BC), jnp.int32),
            pltpu.VMEM((8, BC), jnp.int32),
            pltpu.VMEM((8, BC), jnp.int32),
            pltpu.VMEM((BC, 128), jnp.float32),
            pltpu.VMEM((BC, 128), jnp.float32),
            pltpu.VMEM((D, BC), jnp.float32),
            pltpu.VMEM((D, BC), jnp.float32),
            pltpu.VMEM((S * D,), jnp.float32),
            pltpu.SemaphoreType.DMA,
            pltpu.SemaphoreType.DMA,
            pltpu.SemaphoreType.DMA,
            pltpu.SemaphoreType.DMA,
        ],
        compiler_params=pltpu.CompilerParams(needs_layout_passes=False),
    )
    def look(x_hbm, tok_hbm, pos_hbm, out_hbm, idx_v, idx2_v, cb_v, rows_a,
             rows_b, out_a, out_b, pos_v, g0, g1, o0, o1):
        wid = _wid()
        lane = jnp.arange(_LANES, dtype=jnp.int32)
        tl = [lane + (g * _LANES) for g in range(n_g)]
        zero = _splat(0)
        pltpu.sync_copy(pos_hbm, pos_v)
        rows = [rows_a, rows_b]
        outs = [out_a, out_b]
        gsems = [g0, g1]
        osems = [o0, o1]

        def task(j, carry):
            t = j * _NW + wid
            s_hi = t // chunks
            b0 = pl.multiple_of((t % chunks) * BC, 128)
            pltpu.sync_copy(x_hbm.at[s_hi, :, pl.ds(b0, BC)], idx_v)

            def halve(g, c):
                r = g // n_g
                gg = g % n_g
                q = gg * _LANES
                raw = idx_v[r, pl.ds(q, _LANES)]
                idx2_v[r, pl.ds(q, _LANES)] = raw >> 1
                cb_v[r, pl.ds(q, _LANES)] = ((raw & 1) << 6) | (
                    (lane + (gg * _LANES)) << 7
                )
                return c

            lax.fori_loop(0, 8 * n_g, halve, 0)

            gathers = [None, None]
            gathers[0] = pltpu.async_copy(
                tok_hbm.at[idx2_v.at[0]], rows[0], gsems[0]
            )
            out_copies = [None, None]

            for s_lo in range(8):
                buf = s_lo % 2
                s = s_hi * 8 + s_lo
                gathers[buf].wait()
                if s_lo + 1 < 8:
                    gathers[1 - buf] = pltpu.async_copy(
                        tok_hbm.at[idx2_v.at[s_lo + 1]],
                        rows[1 - buf],
                        gsems[1 - buf],
                    )
                if out_copies[buf] is not None:
                    out_copies[buf].wait()

                rbuf = rows[buf]
                obuf = outs[buf]
                sD = s * D

                def col(i, c):
                    d0 = i * 4
                    pds = [
                        plsc.load_gather(pos_v, [_splat(sD + d0 + u)])
                        for u in range(4)
                    ]
                    dvs = [_splat(d0 + u) for u in range(4)]
                    for g in range(n_g):
                        cbg = cb_v[s_lo, pl.ds(g * _LANES, _LANES)]
                        for u in range(4):
                            vec = plsc.load_gather(rbuf, [zero, cbg + dvs[u]])
                            obuf[d0 + u, pl.ds(g * _LANES, _LANES)] = vec + pds[u]
                    return c

                lax.fori_loop(0, D // 4, col, 0)
                out_copies[buf] = pltpu.async_copy(
                    obuf, out_hbm.at[s, :, pl.ds(b0, BC)], osems[buf]
                )

            for cp in out_copies:
                cp.wait()
            return carry

        lax.fori_loop(0, n_tasks // _NW, task, 0)

    return look


def kernel(x, token_table, pos_table):
    B, S = x.shape
    V, D = token_table.shape
    tok2 = token_table.reshape(V // 2, 2 * D)
    pos_flat = pos_table.reshape(-1)
    out_t = _make_lookup(B, S, D, V, 128)(
        x.T.reshape(S // 8, 8, B).astype(jnp.int32), tok2, pos_flat
    )
    return out_t.transpose(2, 0, 1)


# speedup vs baseline: 1.4987x; 1.4987x over previous
import functools

import jax
import jax.numpy as jnp
from jax import lax
from jax.experimental import pallas as pl
from jax.experimental.pallas import tpu as pltpu
from jax.experimental.pallas import tpu_sc as plsc

_NW = 32
_LANES = 16


def _wid():
    return lax.axis_index("s") * 2 + lax.axis_index("c")


def _splat(value):
    return jnp.full((_LANES,), value, dtype=jnp.int32)


@functools.lru_cache(maxsize=None)
def _make_lookup(B, S, D, V, BC):
    assert BC == 128 and D == 64 and S % 8 == 0 and B % BC == 0 and V % 2 == 0
    chunks = B // BC
    n_tasks = (S // 8) * chunks
    assert n_tasks % _NW == 0
    n_g = BC // _LANES

    mesh = plsc.VectorSubcoreMesh(core_axis_name="c", subcore_axis_name="s")

    @functools.partial(
        pl.kernel,
        mesh=mesh,
        out_type=jax.ShapeDtypeStruct((S, D, B), jnp.float32),
        scratch_types=[
            pltpu.VMEM((8, BC), jnp.int32),
            pltpu.VMEM((8, BC), jnp.int32),
            pltpu.VMEM((8, BC), jnp.int32),
            pltpu.VMEM((BC, 128), jnp.float32),
            pltpu.VMEM((BC, 128), jnp.float32),
            pltpu.VMEM((D, BC), jnp.float32),
            pltpu.VMEM((D, BC), jnp.float32),
            pltpu.VMEM((S * D,), jnp.float32),
            pltpu.SemaphoreType.DMA,
            pltpu.SemaphoreType.DMA,
            pltpu.SemaphoreType.DMA,
            pltpu.SemaphoreType.DMA,
        ],
        compiler_params=pltpu.CompilerParams(needs_layout_passes=False),
    )
    def look(x_hbm, tok_hbm, pos_hbm, out_hbm, idx_v, idx2_v, cb_v, rows_a,
             rows_b, out_a, out_b, pos_v, g0, g1, o0, o1):
        wid = _wid()
        lane = jnp.arange(_LANES, dtype=jnp.int32)
        tl = [lane + (g * _LANES) for g in range(n_g)]
        zero = _splat(0)
        pltpu.sync_copy(pos_hbm, pos_v)
        rows = [rows_a, rows_b]
        outs = [out_a, out_b]
        gsems = [g0, g1]
        osems = [o0, o1]

        def task(j, carry):
            t = j * _NW + wid
            s_hi = t // chunks
            b0 = pl.multiple_of((t % chunks) * BC, 128)
            pltpu.sync_copy(x_hbm.at[s_hi, :, pl.ds(b0, BC)], idx_v)

            def halve(g, c):
                r = g // n_g
                gg = g % n_g
                q = gg * _LANES
                raw = idx_v[r, pl.ds(q, _LANES)]
                idx2_v[r, pl.ds(q, _LANES)] = raw >> 1
                cb_v[r, pl.ds(q, _LANES)] = ((raw & 1) << 6) | (
                    (lane + (gg * _LANES)) << 7
                )
                return c

            lax.fori_loop(0, 8 * n_g, halve, 0)

            gathers = [None, None]
            gathers[0] = pltpu.async_copy(
                tok_hbm.at[idx2_v.at[0]], rows[0], gsems[0]
            )
            out_copies = [None, None]

            for s_lo in range(8):
                buf = s_lo % 2
                s = s_hi * 8 + s_lo
                gathers[buf].wait()
                if s_lo + 1 < 8:
                    gathers[1 - buf] = pltpu.async_copy(
                        tok_hbm.at[idx2_v.at[s_lo + 1]],
                        rows[1 - buf],
                        gsems[1 - buf],
                    )
                if out_copies[buf] is not None:
                    out_copies[buf].wait()

                rbuf = rows[buf]
                obuf = outs[buf]
                sD = s * D

                @plsc.parallel_loop(0, D, unroll=4)
                def col(d):
                    pd = plsc.load_gather(pos_v, [_splat(sD + d)])
                    dv = _splat(d)
                    for g in range(n_g):
                        cbg = cb_v[s_lo, pl.ds(g * _LANES, _LANES)]
                        vec = plsc.load_gather(rbuf, [zero, cbg + dv])
                        obuf[d, pl.ds(g * _LANES, _LANES)] = vec + pd
                out_copies[buf] = pltpu.async_copy(
                    obuf, out_hbm.at[s, :, pl.ds(b0, BC)], osems[buf]
                )

            for cp in out_copies:
                cp.wait()
            return carry

        lax.fori_loop(0, n_tasks // _NW, task, 0)

    return look


def kernel(x, token_table, pos_table):
    B, S = x.shape
    V, D = token_table.shape
    tok2 = token_table.reshape(V // 2, 2 * D)
    pos_flat = pos_table.reshape(-1)
    out_t = _make_lookup(B, S, D, V, 128)(
        x.T.reshape(S // 8, 8, B).astype(jnp.int32), tok2, pos_flat
    )
    return out_t.transpose(2, 0, 1)


# speedup vs baseline: 1.5055x; 1.0046x over previous
import functools

import jax
import jax.numpy as jnp
from jax import lax
from jax.experimental import pallas as pl
from jax.experimental.pallas import tpu as pltpu
from jax.experimental.pallas import tpu_sc as plsc

_NW = 32
_LANES = 16


def _wid():
    return lax.axis_index("s") * 2 + lax.axis_index("c")


def _splat(value):
    return jnp.full((_LANES,), value, dtype=jnp.int32)


@functools.lru_cache(maxsize=None)
def _make_lookup(B, S, D, V, BC):
    assert BC == 128 and D == 64 and S % 8 == 0 and B % BC == 0 and V % 2 == 0
    chunks = B // BC
    n_tasks = (S // 8) * chunks
    assert n_tasks % _NW == 0
    n_g = BC // _LANES

    mesh = plsc.VectorSubcoreMesh(core_axis_name="c", subcore_axis_name="s")

    @functools.partial(
        pl.kernel,
        mesh=mesh,
        out_type=jax.ShapeDtypeStruct((S, D, B), jnp.float32),
        scratch_types=[
            pltpu.VMEM((8, BC), jnp.int32),
            pltpu.VMEM((8, BC), jnp.int32),
            pltpu.VMEM((8, BC), jnp.int32),
            pltpu.VMEM((BC, 128), jnp.float32),
            pltpu.VMEM((BC, 128), jnp.float32),
            pltpu.VMEM((D, BC), jnp.float32),
            pltpu.VMEM((D, BC), jnp.float32),
            pltpu.VMEM((S * D,), jnp.float32),
            pltpu.SemaphoreType.DMA,
            pltpu.SemaphoreType.DMA,
            pltpu.SemaphoreType.DMA,
            pltpu.SemaphoreType.DMA,
        ],
        compiler_params=pltpu.CompilerParams(needs_layout_passes=False),
    )
    def look(x_hbm, tok_hbm, pos_hbm, out_hbm, idx_v, idx2_v, cb_v, rows_a,
             rows_b, out_a, out_b, pos_v, g0, g1, o0, o1):
        wid = _wid()
        lane = jnp.arange(_LANES, dtype=jnp.int32)
        tl = [lane + (g * _LANES) for g in range(n_g)]
        zero = _splat(0)
        pltpu.sync_copy(pos_hbm, pos_v)
        rows = [rows_a, rows_b]
        outs = [out_a, out_b]
        gsems = [g0, g1]
        osems = [o0, o1]

        def task(j, carry):
            t = j * _NW + wid
            s_hi = t // chunks
            b0 = pl.multiple_of((t % chunks) * BC, 128)
            pltpu.sync_copy(x_hbm.at[s_hi, :, pl.ds(b0, BC)], idx_v)

            @plsc.parallel_loop(0, 8 * n_g, unroll=4)
            def halve(g):
                r = g // n_g
                gg = g % n_g
                q = gg * _LANES
                raw = idx_v[r, pl.ds(q, _LANES)]
                idx2_v[r, pl.ds(q, _LANES)] = raw >> 1
                cb_v[r, pl.ds(q, _LANES)] = ((raw & 1) << 6) | (
                    (lane + (gg * _LANES)) << 7
                )

            gathers = [None, None]
            gathers[0] = pltpu.async_copy(
                tok_hbm.at[idx2_v.at[0]], rows[0], gsems[0]
            )
            out_copies = [None, None]

            for s_lo in range(8):
                buf = s_lo % 2
                s = s_hi * 8 + s_lo
                gathers[buf].wait()
                if s_lo + 1 < 8:
                    gathers[1 - buf] = pltpu.async_copy(
                        tok_hbm.at[idx2_v.at[s_lo + 1]],
                        rows[1 - buf],
                        gsems[1 - buf],
                    )
                if out_copies[buf] is not None:
                    out_copies[buf].wait()

                rbuf = rows[buf]
                obuf = outs[buf]
                sD = s * D

                @plsc.parallel_loop(0, D, unroll=8)
                def col(d):
                    pd = plsc.load_gather(pos_v, [_splat(sD + d)])
                    dv = _splat(d)
                    for g in range(n_g):
                        cbg = cb_v[s_lo, pl.ds(g * _LANES, _LANES)]
                        vec = plsc.load_gather(rbuf, [zero, cbg + dv])
                        obuf[d, pl.ds(g * _LANES, _LANES)] = vec + pd
                out_copies[buf] = pltpu.async_copy(
                    obuf, out_hbm.at[s, :, pl.ds(b0, BC)], osems[buf]
                )

            for cp in out_copies:
                cp.wait()
            return carry

        lax.fori_loop(0, n_tasks // _NW, task, 0)

    return look


def kernel(x, token_table, pos_table):
    B, S = x.shape
    V, D = token_table.shape
    tok2 = token_table.reshape(V // 2, 2 * D)
    pos_flat = pos_table.reshape(-1)
    out_t = _make_lookup(B, S, D, V, 128)(
        x.T.reshape(S // 8, 8, B).astype(jnp.int32), tok2, pos_flat
    )
    return out_t.transpose(2, 0, 1)
